# baseline (speedup 1.0000x reference)
import jax
import jax.numpy as jnp
from jax.experimental import pallas as pl

_HI = jax.lax.Precision.HIGHEST


def _bank_kernel(x_ref, b2_ref, k1_ref, k2_ref, k3_ref, xfeat_ref):
    x = x_ref[...]
    x_feat = jnp.sum(x, axis=1) * 0.125
    xfeat_ref[...] = x_feat
    keys_un = jnp.dot(x_feat, b2_ref[...],
                      preferred_element_type=jnp.float32, precision=_HI)
    ss = jnp.sum(keys_un * keys_un, axis=-1, keepdims=True)
    nrm = jnp.maximum(jnp.sqrt(ss), 1e-12)
    keys = keys_un / nrm
    k1_ref[...] = keys
    k2_ref[...] = keys
    k3_ref[...] = keys


def _ymean_kernel(y_ref, ym_ref):
    ym_ref[...] = jnp.sum(y_ref[...], axis=1) * 0.125


def kernel(all_x, all_y, w_ext, b_ext, w_cp, b_cp, W_enc):
    M, T, N = all_x.shape
    P = all_y.shape[1]
    BINS, D = W_enc.shape

    xt = jnp.transpose(all_x, (0, 2, 1))
    yt = jnp.transpose(all_y, (1, 2, 0))

    B2 = jnp.repeat(W_enc, T // BINS, axis=0) / (T // BINS)

    BP = 16
    ym_t = pl.pallas_call(
        _ymean_kernel,
        grid=(P // BP,),
        in_specs=[pl.BlockSpec((BP, N, M), lambda i: (i, 0, 0))],
        out_specs=pl.BlockSpec((BP, M), lambda i: (i, 0)),
        out_shape=jax.ShapeDtypeStruct((P, M), jnp.float32),
    )(yt)

    BM = 1024
    grid = (M // BM,)
    k1, k2, k3, x_feat = pl.pallas_call(
        _bank_kernel,
        grid=grid,
        in_specs=[
            pl.BlockSpec((BM, N, T), lambda i: (i, 0, 0)),
            pl.BlockSpec((T, D), lambda i: (0, 0)),
        ],
        out_specs=[
            pl.BlockSpec((BM, D), lambda i: (i, 0)),
            pl.BlockSpec((BM, D), lambda i: (i, 0)),
            pl.BlockSpec((BM, D), lambda i: (i, 0)),
            pl.BlockSpec((BM, T), lambda i: (i, 0)),
        ],
        out_shape=[
            jax.ShapeDtypeStruct((M, D), jnp.float32),
            jax.ShapeDtypeStruct((M, D), jnp.float32),
            jax.ShapeDtypeStruct((M, D), jnp.float32),
            jax.ShapeDtypeStruct((M, T), jnp.float32),
        ],
    )(xt, B2)
    ym = ym_t.T

    extreme_probs = jax.nn.sigmoid(x_feat @ w_ext + b_ext)
    near_end_scores = jax.nn.sigmoid(x_feat[:, -64:] @ w_cp + b_cp)
    labels = jnp.zeros((M,), dtype=jnp.int32)
    labels = jnp.where(extreme_probs > 0.5, jnp.int32(1), labels)
    labels = jnp.where(near_end_scores > 0.5, jnp.int32(2), labels)
    return (k1, k2, k3, ym, labels)


# speedup vs baseline: 1.0440x; 1.0440x over previous
import jax
import jax.numpy as jnp
from jax.experimental import pallas as pl

_HI = jax.lax.Precision.HIGHEST


def _bank_kernel(x_ref, y_ref, b2_ref, k1_ref, k2_ref, k3_ref,
                 ym_ref, xfeat_ref):
    x = x_ref[...]
    x_feat = jnp.sum(x, axis=1) * 0.125
    xfeat_ref[...] = x_feat
    keys_un = jnp.dot(x_feat, b2_ref[...],
                      preferred_element_type=jnp.float32, precision=_HI)
    ss = jnp.sum(keys_un * keys_un, axis=-1, keepdims=True)
    nrm = jnp.maximum(jnp.sqrt(ss), 1e-12)
    keys = keys_un / nrm
    k1_ref[...] = keys
    k2_ref[...] = keys
    k3_ref[...] = keys
    y = y_ref[...]
    ym_ref[...] = jnp.sum(y, axis=1) * 0.125


def kernel(all_x, all_y, w_ext, b_ext, w_cp, b_cp, W_enc):
    M, T, N = all_x.shape
    P = all_y.shape[1]
    BINS, D = W_enc.shape

    xt = jnp.transpose(all_x, (0, 2, 1))
    yt = jnp.transpose(all_y, (1, 2, 0))

    B2 = jnp.repeat(W_enc, T // BINS, axis=0) / (T // BINS)

    BM = 1024
    grid = (M // BM,)
    k1, k2, k3, ym_t, x_feat = pl.pallas_call(
        _bank_kernel,
        grid=grid,
        in_specs=[
            pl.BlockSpec((BM, N, T), lambda i: (i, 0, 0)),
            pl.BlockSpec((P, N, BM), lambda i: (0, 0, i)),
            pl.BlockSpec((T, D), lambda i: (0, 0)),
        ],
        out_specs=[
            pl.BlockSpec((BM, D), lambda i: (i, 0)),
            pl.BlockSpec((BM, D), lambda i: (i, 0)),
            pl.BlockSpec((BM, D), lambda i: (i, 0)),
            pl.BlockSpec((P, BM), lambda i: (0, i)),
            pl.BlockSpec((BM, T), lambda i: (i, 0)),
        ],
        out_shape=[
            jax.ShapeDtypeStruct((M, D), jnp.float32),
            jax.ShapeDtypeStruct((M, D), jnp.float32),
            jax.ShapeDtypeStruct((M, D), jnp.float32),
            jax.ShapeDtypeStruct((P, M), jnp.float32),
            jax.ShapeDtypeStruct((M, T), jnp.float32),
        ],
    )(xt, yt, B2)
    ym = ym_t.T

    extreme_probs = jax.nn.sigmoid(x_feat @ w_ext + b_ext)
    near_end_scores = jax.nn.sigmoid(x_feat[:, -64:] @ w_cp + b_cp)
    labels = jnp.zeros((M,), dtype=jnp.int32)
    labels = jnp.where(extreme_probs > 0.5, jnp.int32(1), labels)
    labels = jnp.where(near_end_scores > 0.5, jnp.int32(2), labels)
    return (k1, k2, k3, ym, labels)
